# baseline (speedup 1.0000x reference)
import jax
import jax.numpy as jnp
from jax import lax
from jax.experimental import pallas as pl
from jax.experimental.pallas import tpu as pltpu
from jax.experimental.pallas import tpu_sc as plsc

N_NODES = 10000
N_EDGES = 320000
D_FEAT = 128

NC = 2
NS = 16
L = 16

DH = D_FEAT // 2
B = 80
EPT = N_EDGES // NS
NCHUNK = EPT // B
NBUF = 5
ZROWS = 25
NPT = N_NODES // NS


def _sc_body(x2_hbm, src_hbm, dst_hbm, val_hbm, out_hbm,
             gidx_v, dst_v, val_v, rows_v, zbuf_v, accum_sh, sem_g, sem_s):
  c = lax.axis_index("c")
  s = lax.axis_index("s")

  def zero_row(r, carry):
    for j in range(DH // L):
      zbuf_v[r, pl.ds(j * L, L)] = jnp.zeros((L,), jnp.float32)
    return carry
  lax.fori_loop(0, ZROWS, zero_row, 0, unroll=2)
  for i in range(NPT // ZROWS):
    pltpu.sync_copy(zbuf_v, accum_sh.at[pl.ds(s * NPT + i * ZROWS, ZROWS)])

  pltpu.sync_copy(src_hbm.at[s], gidx_v)
  pltpu.sync_copy(dst_hbm.at[s], dst_v)
  pltpu.sync_copy(val_hbm.at[s], val_v)

  def to_gidx(i, carry):
    v = gidx_v[pl.ds(i * L, L)]
    gidx_v[pl.ds(i * L, L)] = v * 2 + c
    return carry
  lax.fori_loop(0, EPT // L, to_gidx, 0, unroll=4)
  plsc.subcore_barrier()

  def gather(k, p):
    return pltpu.make_async_copy(
        x2_hbm.at[gidx_v.at[pl.ds(k * B, B)]], rows_v.at[p], sem_g)

  def scatter(k, p):
    return pltpu.make_async_copy(rows_v.at[p], accum_sh.at[dst_v.at[k]], sem_s)

  gather(0, 0).start()
  gather(1, 1).start()
  gather(2, 2).start()

  def chunk(k, carry):
    p = lax.rem(k, NBUF)
    pn = lax.rem(k + 3, NBUF)
    gather(k, p).wait()

    @pl.when(k >= 2)
    def _():
      scatter(k - 2, pn).wait()

    @pl.when(k + 3 < NCHUNK)
    def _():
      gather(k + 3, pn).start()

    for g in range(B // L):
      vals16 = val_v[pl.ds(k * B + g * L, L)]
      for j in range(L):
        v = vals16[jnp.full((L,), j, jnp.int32)]
        e = g * L + j
        for f in range(DH // L):
          sl = pl.ds(f * L, L)
          rows_v[p, e, sl] = rows_v[p, e, sl] * v

    scatter(k, p).start(add=True)
    return carry

  lax.fori_loop(0, NCHUNK, chunk, 0)
  scatter(NCHUNK - 2, lax.rem(NCHUNK - 2, NBUF)).wait()
  scatter(NCHUNK - 1, lax.rem(NCHUNK - 1, NBUF)).wait()

  plsc.subcore_barrier()
  pltpu.sync_copy(accum_sh.at[pl.ds(s * NPT, NPT)],
                  out_hbm.at[pl.ds(s * NPT, NPT), pl.ds(c * DH, DH)])


@jax.jit
def kernel(x, edge_index, edge_values):
  x2 = x.reshape(2 * N_NODES, DH)
  src = edge_index[1].astype(jnp.int32).reshape(NS, EPT)
  dst = edge_index[0].astype(jnp.int32).reshape(NS, NCHUNK, B)
  val = edge_values.reshape(NS, EPT)

  mesh = plsc.VectorSubcoreMesh(core_axis_name="c", subcore_axis_name="s",
                                num_cores=NC, num_subcores=NS)
  return pl.kernel(
      _sc_body,
      out_type=jax.ShapeDtypeStruct((N_NODES, D_FEAT), jnp.float32),
      mesh=mesh,
      compiler_params=pltpu.CompilerParams(use_tc_tiling_on_sc=False,
                                           needs_layout_passes=False),
      scratch_types=[
          pltpu.VMEM((EPT,), jnp.int32),
          pltpu.VMEM((NCHUNK, B), jnp.int32),
          pltpu.VMEM((EPT,), jnp.float32),
          pltpu.VMEM((NBUF, B, DH), jnp.float32),
          pltpu.VMEM((ZROWS, DH), jnp.float32),
          pltpu.VMEM_SHARED((N_NODES, DH), jnp.float32),
          pltpu.SemaphoreType.DMA,
          pltpu.SemaphoreType.DMA,
      ],
  )(x2, src, dst, val)


# speedup vs baseline: 1.0096x; 1.0096x over previous
import jax
import jax.numpy as jnp
from jax import lax
from jax.experimental import pallas as pl
from jax.experimental.pallas import tpu as pltpu
from jax.experimental.pallas import tpu_sc as plsc

N_NODES = 10000
N_EDGES = 320000
D_FEAT = 128

NC = 2
NS = 16
L = 16

DH = D_FEAT // 2
B = 80
EPT = N_EDGES // NS
NCHUNK = EPT // B
NBUF = 5
ZROWS = 25
NPT = N_NODES // NS


def _sc_body(x2_hbm, src_hbm, dst_hbm, val_hbm, out_hbm,
             gidx_v, dst_v, val_v, rows_v, zbuf_v, accum_sh, sem_g, sem_s):
  c = lax.axis_index("c")
  s = lax.axis_index("s")

  def zero_row(r, carry):
    for j in range(DH // L):
      zbuf_v[r, pl.ds(j * L, L)] = jnp.zeros((L,), jnp.float32)
    return carry
  lax.fori_loop(0, ZROWS, zero_row, 0, unroll=2)
  for i in range(NPT // ZROWS):
    pltpu.sync_copy(zbuf_v, accum_sh.at[pl.ds(s * NPT + i * ZROWS, ZROWS)])

  st1 = pltpu.make_async_copy(src_hbm.at[s], gidx_v, sem_g)
  st2 = pltpu.make_async_copy(dst_hbm.at[s], dst_v, sem_g)
  st3 = pltpu.make_async_copy(val_hbm.at[s], val_v, sem_g)
  st1.start(); st2.start(); st3.start()
  st1.wait(); st2.wait(); st3.wait()

  def to_gidx(i, carry):
    v = gidx_v[pl.ds(i * L, L)]
    gidx_v[pl.ds(i * L, L)] = v * 2 + c
    return carry
  lax.fori_loop(0, EPT // L, to_gidx, 0, unroll=4)
  plsc.subcore_barrier()

  def gather(k, p):
    return pltpu.make_async_copy(
        x2_hbm.at[gidx_v.at[pl.ds(k * B, B)]], rows_v.at[p], sem_g)

  def scatter(k, p):
    return pltpu.make_async_copy(rows_v.at[p], accum_sh.at[dst_v.at[k]], sem_s)

  gather(0, 0).start()
  gather(1, 1).start()
  gather(2, 2).start()

  def chunk(k, carry):
    p = lax.rem(k, NBUF)
    pn = lax.rem(k + 3, NBUF)
    gather(k, p).wait()

    @pl.when(k >= 2)
    def _():
      scatter(k - 2, pn).wait()

    @pl.when(k + 3 < NCHUNK)
    def _():
      gather(k + 3, pn).start()

    for g in range(B // L):
      vals16 = val_v[pl.ds(k * B + g * L, L)]
      for j in range(L):
        v = vals16[jnp.full((L,), j, jnp.int32)]
        e = g * L + j
        for f in range(DH // L):
          sl = pl.ds(f * L, L)
          rows_v[p, e, sl] = rows_v[p, e, sl] * v

    scatter(k, p).start(add=True)
    return carry

  lax.fori_loop(0, NCHUNK, chunk, 0)
  scatter(NCHUNK - 2, lax.rem(NCHUNK - 2, NBUF)).wait()
  scatter(NCHUNK - 1, lax.rem(NCHUNK - 1, NBUF)).wait()

  plsc.subcore_barrier()
  pltpu.sync_copy(accum_sh.at[pl.ds(s * NPT, NPT)],
                  out_hbm.at[pl.ds(s * NPT, NPT), pl.ds(c * DH, DH)])


@jax.jit
def kernel(x, edge_index, edge_values):
  x2 = x.reshape(2 * N_NODES, DH)
  src = edge_index[1].astype(jnp.int32).reshape(NS, EPT)
  dst = edge_index[0].astype(jnp.int32).reshape(NS, NCHUNK, B)
  val = edge_values.reshape(NS, EPT)

  mesh = plsc.VectorSubcoreMesh(core_axis_name="c", subcore_axis_name="s",
                                num_cores=NC, num_subcores=NS)
  return pl.kernel(
      _sc_body,
      out_type=jax.ShapeDtypeStruct((N_NODES, D_FEAT), jnp.float32),
      mesh=mesh,
      compiler_params=pltpu.CompilerParams(use_tc_tiling_on_sc=False,
                                           needs_layout_passes=False),
      scratch_types=[
          pltpu.VMEM((EPT,), jnp.int32),
          pltpu.VMEM((NCHUNK, B), jnp.int32),
          pltpu.VMEM((EPT,), jnp.float32),
          pltpu.VMEM((NBUF, B, DH), jnp.float32),
          pltpu.VMEM((ZROWS, DH), jnp.float32),
          pltpu.VMEM_SHARED((N_NODES, DH), jnp.float32),
          pltpu.SemaphoreType.DMA,
          pltpu.SemaphoreType.DMA,
      ],
  )(x2, src, dst, val)


# speedup vs baseline: 1.0295x; 1.0197x over previous
import jax
import jax.numpy as jnp
from jax import lax
from jax.experimental import pallas as pl
from jax.experimental.pallas import tpu as pltpu
from jax.experimental.pallas import tpu_sc as plsc

N_NODES = 10000
N_EDGES = 320000
D_FEAT = 128

NC = 2
NS = 16
L = 16

DH = D_FEAT // 2
B = 80
EPT = N_EDGES // NS
NCHUNK = EPT // B
NBUF = 5
ZROWS = 25
NPT = N_NODES // NS


def _sc_body(x2_hbm, src_hbm, dst_hbm, val_hbm, out_hbm,
             gidx_v, dst_v, val_v, rows_v, zbuf_v, accum_sh, sem_g, sem_s):
  c = lax.axis_index("c")
  s = lax.axis_index("s")

  st1 = pltpu.make_async_copy(src_hbm.at[s], gidx_v, sem_g)
  st2 = pltpu.make_async_copy(dst_hbm.at[s], dst_v, sem_g)
  st3 = pltpu.make_async_copy(val_hbm.at[s], val_v, sem_g)
  st1.start(); st2.start(); st3.start()

  def zero_row(r, carry):
    for j in range(DH // L):
      zbuf_v[r, pl.ds(j * L, L)] = jnp.zeros((L,), jnp.float32)
    return carry
  lax.fori_loop(0, ZROWS, zero_row, 0, unroll=2)
  for i in range(NPT // ZROWS):
    pltpu.sync_copy(zbuf_v, accum_sh.at[pl.ds(s * NPT + i * ZROWS, ZROWS)])

  st1.wait(); st2.wait(); st3.wait()

  def to_gidx(i, carry):
    v = gidx_v[pl.ds(i * L, L)]
    gidx_v[pl.ds(i * L, L)] = v * 2 + c
    return carry
  lax.fori_loop(0, EPT // L, to_gidx, 0, unroll=4)
  plsc.subcore_barrier()

  def gather(k, p):
    return pltpu.make_async_copy(
        x2_hbm.at[gidx_v.at[pl.ds(k * B, B)]], rows_v.at[p], sem_g)

  def scatter(k, p):
    return pltpu.make_async_copy(rows_v.at[p], accum_sh.at[dst_v.at[k]], sem_s)

  gather(0, 0).start()
  gather(1, 1).start()
  gather(2, 2).start()

  def chunk(k, carry):
    p = lax.rem(k, NBUF)
    pn = lax.rem(k + 3, NBUF)
    gather(k, p).wait()

    @pl.when(k >= 2)
    def _():
      scatter(k - 2, pn).wait()

    @pl.when(k + 3 < NCHUNK)
    def _():
      gather(k + 3, pn).start()

    for g in range(B // L):
      vals16 = val_v[pl.ds(k * B + g * L, L)]
      for j in range(L):
        v = vals16[jnp.full((L,), j, jnp.int32)]
        e = g * L + j
        for f in range(DH // L):
          sl = pl.ds(f * L, L)
          rows_v[p, e, sl] = rows_v[p, e, sl] * v

    scatter(k, p).start(add=True)
    return carry

  lax.fori_loop(0, NCHUNK, chunk, 0)
  scatter(NCHUNK - 2, lax.rem(NCHUNK - 2, NBUF)).wait()
  scatter(NCHUNK - 1, lax.rem(NCHUNK - 1, NBUF)).wait()

  plsc.subcore_barrier()
  pltpu.sync_copy(accum_sh.at[pl.ds(s * NPT, NPT)],
                  out_hbm.at[pl.ds(s * NPT, NPT), pl.ds(c * DH, DH)])


@jax.jit
def kernel(x, edge_index, edge_values):
  x2 = x.reshape(2 * N_NODES, DH)
  src = edge_index[1].astype(jnp.int32).reshape(NS, EPT)
  dst = edge_index[0].astype(jnp.int32).reshape(NS, NCHUNK, B)
  val = edge_values.reshape(NS, EPT)

  mesh = plsc.VectorSubcoreMesh(core_axis_name="c", subcore_axis_name="s",
                                num_cores=NC, num_subcores=NS)
  return pl.kernel(
      _sc_body,
      out_type=jax.ShapeDtypeStruct((N_NODES, D_FEAT), jnp.float32),
      mesh=mesh,
      compiler_params=pltpu.CompilerParams(use_tc_tiling_on_sc=False,
                                           needs_layout_passes=False),
      scratch_types=[
          pltpu.VMEM((EPT,), jnp.int32),
          pltpu.VMEM((NCHUNK, B), jnp.int32),
          pltpu.VMEM((EPT,), jnp.float32),
          pltpu.VMEM((NBUF, B, DH), jnp.float32),
          pltpu.VMEM((ZROWS, DH), jnp.float32),
          pltpu.VMEM_SHARED((N_NODES, DH), jnp.float32),
          pltpu.SemaphoreType.DMA,
          pltpu.SemaphoreType.DMA,
      ],
  )(x2, src, dst, val)
